# baseline (speedup 1.0000x reference)
import functools

import jax
import jax.numpy as jnp
from jax import lax
from jax.experimental import pallas as pl
from jax.experimental.pallas import tpu as pltpu
from jax.experimental.pallas import tpu_sc as plsc

D_MODEL = 64
MAX_LEN = 200
BATCH = 4096
NUM_WORKERS = 32
BB = 128
NSLAB = 4
SLAB = BATCH // NSLAB
BLKS = SLAB // BB
TROWS = MAX_LEN * BLKS // NUM_WORKERS
NBUF = 5
TC_BB = 128
T_BLK = 40


def _pos_encoding():
    even_i = jnp.arange(0, D_MODEL, 2).astype(jnp.float32)
    denominator = jnp.power(10000.0, even_i / D_MODEL)
    position = jnp.arange(MAX_LEN, dtype=jnp.float32).reshape(MAX_LEN, 1)
    even_pe = jnp.sin(position / denominator)
    odd_pe = jnp.cos(position / denominator)
    return jnp.stack([even_pe, odd_pe], axis=2).reshape(MAX_LEN, D_MODEL)


def _sc_gather(idx_t3, table_wide, slab):
    mesh = plsc.VectorSubcoreMesh(core_axis_name="c", subcore_axis_name="s")

    @functools.partial(
        pl.kernel,
        mesh=mesh,
        out_type=jax.ShapeDtypeStruct((MAX_LEN, SLAB, 2 * D_MODEL), jnp.float32),
        scratch_types=[
            pltpu.VMEM((MAX_LEN, BB), jnp.int32),
            pltpu.VMEM((NBUF, BB, 2 * D_MODEL), jnp.float32),
            pltpu.SemaphoreType.DMA((NBUF,)),
            pltpu.SemaphoreType.DMA((NBUF,)),
        ],
    )
    def k(idx_hbm, table_hbm, out_hbm, idx_v, bufs, gsem, osem):
        wid = lax.axis_index("s") * 2 + lax.axis_index("c")
        blk = wid % BLKS
        t0 = (wid // BLKS) * TROWS
        b0 = blk * BB
        pltpu.sync_copy(idx_hbm.at[slab * BLKS + blk], idx_v)

        def start_gather(t, s):
            pltpu.async_copy(
                table_hbm.at[idx_v.at[t0 + t]], bufs.at[s], gsem.at[s])

        def finish(t, s):
            pltpu.make_async_copy(
                table_hbm.at[pl.ds(0, BB)], bufs.at[s], gsem.at[s]).wait()
            pltpu.async_copy(
                bufs.at[s], out_hbm.at[t0 + t, pl.ds(b0, BB)], osem.at[s])

        def wait_out(s):
            pltpu.make_async_copy(
                bufs.at[s], out_hbm.at[0, pl.ds(b0, BB)], osem.at[s]).wait()

        def outer(io, carry):
            for s in range(NBUF):
                t = io * NBUF + s

                @pl.when(io >= 1)
                def _():
                    wait_out(s)

                start_gather(t, s)
                if s == 0:
                    @pl.when(io >= 1)
                    def _():
                        finish(io * NBUF - 1, NBUF - 1)
                else:
                    finish(t - 1, s - 1)
            return carry

        lax.fori_loop(0, TROWS // NBUF, outer, 0)
        finish(TROWS - 1, NBUF - 1)
        for s in range(NBUF):
            wait_out(s)

    return k(idx_t3, table_wide)


def _tc_finish(inter, pe, slab, prev_out):
    def body(inter_ref, pe_ref, *rest):
        o_ref = rest[-1]
        x = inter_ref[...][:, :, :D_MODEL]
        xt = jnp.transpose(x, (0, 2, 1))
        o_ref[...] = xt + pe_ref[...][:, :, None]

    in_specs = [
        pl.BlockSpec((T_BLK, TC_BB, 2 * D_MODEL), lambda i, j: (j, i, 0)),
        pl.BlockSpec((T_BLK, D_MODEL), lambda i, j: (j, 0)),
    ]
    operands = [inter, pe]
    aliases = {}
    if prev_out is not None:
        in_specs.append(pl.BlockSpec(memory_space=pl.ANY))
        operands.append(prev_out)
        aliases = {2: 0}

    return pl.pallas_call(
        body,
        grid=(SLAB // TC_BB, MAX_LEN // T_BLK),
        in_specs=in_specs,
        out_specs=pl.BlockSpec(
            (T_BLK, D_MODEL, TC_BB), lambda i, j: (j, 0, i + slab * BLKS)),
        out_shape=jax.ShapeDtypeStruct((MAX_LEN, D_MODEL, BATCH), jnp.float32),
        input_output_aliases=aliases,
    )(*operands)


def kernel(indices, table):
    table_wide = jnp.pad(table, ((0, 0), (0, D_MODEL)))
    idx_t3 = indices.T.reshape(MAX_LEN, NUM_WORKERS, BB).transpose(1, 0, 2)
    pe = _pos_encoding()

    out_t = None
    for slab in range(NSLAB):
        inter = _sc_gather(idx_t3, table_wide, slab)
        out_t = _tc_finish(inter, pe, slab, out_t)
    return out_t.transpose(2, 0, 1)


# speedup vs baseline: 1.0072x; 1.0072x over previous
import functools

import jax
import jax.numpy as jnp
from jax import lax
from jax.experimental import pallas as pl
from jax.experimental.pallas import tpu as pltpu
from jax.experimental.pallas import tpu_sc as plsc

D_MODEL = 64
MAX_LEN = 200
BATCH = 4096
NUM_WORKERS = 32
BB = 128
NSLAB = 4
SLAB = BATCH // NSLAB
BLKS = SLAB // BB
TROWS = MAX_LEN * BLKS // NUM_WORKERS
NBUF = 5
TC_BB = 128
T_BLK = 40


def _pos_encoding():
    even_i = jnp.arange(0, D_MODEL, 2).astype(jnp.float32)
    denominator = jnp.power(10000.0, even_i / D_MODEL)
    position = jnp.arange(MAX_LEN, dtype=jnp.float32).reshape(MAX_LEN, 1)
    even_pe = jnp.sin(position / denominator)
    odd_pe = jnp.cos(position / denominator)
    return jnp.stack([even_pe, odd_pe], axis=2).reshape(MAX_LEN, D_MODEL)


def _sc_gather(idx_t3, table_wide, slab):
    mesh = plsc.VectorSubcoreMesh(core_axis_name="c", subcore_axis_name="s")

    @functools.partial(
        pl.kernel,
        mesh=mesh,
        out_type=jax.ShapeDtypeStruct((MAX_LEN, SLAB, 2 * D_MODEL), jnp.float32),
        scratch_types=[
            pltpu.VMEM((MAX_LEN, BB), jnp.int32),
            pltpu.VMEM((NBUF, BB, 2 * D_MODEL), jnp.float32),
            pltpu.SemaphoreType.DMA((NBUF,)),
            pltpu.SemaphoreType.DMA((NBUF,)),
        ],
    )
    def k(idx_hbm, table_hbm, out_hbm, idx_v, bufs, gsem, osem):
        wid = lax.axis_index("s") * 2 + lax.axis_index("c")
        blk = wid % BLKS
        t0 = (wid // BLKS) * TROWS
        b0 = blk * BB
        pltpu.sync_copy(idx_hbm.at[slab * BLKS + blk], idx_v)

        def start_gather(t, s):
            pltpu.async_copy(
                table_hbm.at[idx_v.at[t0 + t]], bufs.at[s], gsem.at[s])

        def finish(t, s):
            pltpu.make_async_copy(
                table_hbm.at[pl.ds(0, BB)], bufs.at[s], gsem.at[s]).wait()
            pltpu.async_copy(
                bufs.at[s], out_hbm.at[t0 + t, pl.ds(b0, BB)], osem.at[s])

        def wait_out(s):
            pltpu.make_async_copy(
                bufs.at[s], out_hbm.at[0, pl.ds(b0, BB)], osem.at[s]).wait()

        def outer(io, carry):
            for s in range(NBUF):
                t = io * NBUF + s

                @pl.when(io >= 1)
                def _():
                    wait_out(s)

                start_gather(t, s)
                if s == 0:
                    @pl.when(io >= 1)
                    def _():
                        finish(io * NBUF - 1, NBUF - 1)
                else:
                    finish(t - 1, s - 1)
            return carry

        lax.fori_loop(0, TROWS // NBUF, outer, 0)
        finish(TROWS - 1, NBUF - 1)
        for s in range(NBUF):
            wait_out(s)

    return k(idx_t3, table_wide)


def _tc_finish(inter, pe, slab, prev_out):
    def body(inter_ref, pe_ref, *rest):
        o_ref = rest[-1]
        x = inter_ref[...][:, :, :D_MODEL]
        xt = jnp.transpose(x, (0, 2, 1))
        o_ref[...] = xt + pe_ref[...][:, :, None]

    in_specs = [
        pl.BlockSpec((T_BLK, TC_BB, 2 * D_MODEL), lambda i, j: (j, i, 0)),
        pl.BlockSpec((T_BLK, D_MODEL), lambda i, j: (j, 0)),
    ]
    operands = [inter, pe]
    aliases = {}
    if prev_out is not None:
        in_specs.append(pl.BlockSpec(memory_space=pl.ANY))
        operands.append(prev_out)
        aliases = {2: 0}

    return pl.pallas_call(
        body,
        grid=(SLAB // TC_BB, MAX_LEN // T_BLK),
        in_specs=in_specs,
        out_specs=pl.BlockSpec(
            (T_BLK, D_MODEL, TC_BB), lambda i, j: (j, 0, i + slab * (SLAB // TC_BB))),
        out_shape=jax.ShapeDtypeStruct((MAX_LEN, D_MODEL, BATCH), jnp.float32),
        input_output_aliases=aliases,
    )(*operands)


def kernel(indices, table):
    table_wide = jnp.pad(table, ((0, 0), (0, D_MODEL)))
    idx_t3 = indices.T.reshape(MAX_LEN, NUM_WORKERS, BB).transpose(1, 0, 2)
    pe = _pos_encoding()

    out_t = None
    for slab in range(NSLAB):
        inter = _sc_gather(idx_t3, table_wide, slab)
        out_t = _tc_finish(inter, pe, slab, out_t)
    return out_t.transpose(2, 0, 1)
